# baseline (speedup 1.0000x reference)
import functools

import jax
import jax.numpy as jnp
from jax import lax
from jax.experimental import pallas as pl
from jax.experimental.pallas import tpu as pltpu
from jax.experimental.pallas import tpu_sc as plsc

_NC = 2
_NS = 16
_NW = _NC * _NS

_CH = 16


def _sc_span_gather(feat, wemb, gs, ge, wd, ends, *, H, WD, n_total):
    OUT_D = 2 * H + WD
    spw = n_total // _NW
    nchunk = spw // _CH
    mesh = plsc.VectorSubcoreMesh(core_axis_name="c", subcore_axis_name="s")

    @functools.partial(
        pl.kernel,
        out_type=jax.ShapeDtypeStruct((n_total, OUT_D), jnp.float32),
        mesh=mesh,
        compiler_params=pltpu.CompilerParams(use_tc_tiling_on_sc=False),
        scratch_types=[
            pltpu.VMEM((_CH,), jnp.int32),
            pltpu.VMEM((_CH,), jnp.int32),
            pltpu.VMEM((_CH,), jnp.int32),
            pltpu.VMEM((_CH,), jnp.int32),
            pltpu.VMEM((_CH, H), jnp.float32),
            pltpu.VMEM((_CH, H), jnp.float32),
            pltpu.VMEM((_CH, WD), jnp.float32),
            pltpu.SemaphoreType.DMA,
            pltpu.SemaphoreType.DMA,
            pltpu.SemaphoreType.DMA,
        ],
    )
    def k(feat_hbm, wemb_hbm, gs_hbm, ge_hbm, wd_hbm, ends_hbm, out_hbm,
          idxs, idxe, idxw, ends_v, sbuf, ebuf, wbuf, sem0, sem1, sem2):
        wid = lax.axis_index("s") * _NC + lax.axis_index("c")
        base = wid * spw

        def chunk_body(c, _):
            cb = base + c * _CH
            pltpu.sync_copy(gs_hbm.at[pl.ds(cb, _CH)], idxs)
            pltpu.sync_copy(ge_hbm.at[pl.ds(cb, _CH)], idxe)
            pltpu.sync_copy(wd_hbm.at[pl.ds(cb, _CH)], idxw)
            pltpu.sync_copy(ends_hbm.at[pl.ds(cb, _CH)], ends_v)
            cs = pltpu.async_copy(feat_hbm.at[idxs], sbuf, sem0)
            ce = pltpu.async_copy(feat_hbm.at[idxe], ebuf, sem1)
            cw = pltpu.async_copy(wemb_hbm.at[idxw], wbuf, sem2)
            cs.wait()
            ce.wait()
            cw.wait()

            evec = ends_v[...]
            for j in range(_CH):
                @pl.when(evec[j] <= 0)
                def _zero(j=j):
                    z = jnp.zeros((16,), jnp.float32)

                    def zrow(kk, _):
                        sbuf[j, pl.ds(kk * 16, 16)] = z
                        ebuf[j, pl.ds(kk * 16, 16)] = z
                        return 0

                    lax.fori_loop(0, H // 16, zrow, 0)

                    def zw(kk, _):
                        wbuf[j, pl.ds(kk * 16, 16)] = z
                        return 0

                    lax.fori_loop(0, WD // 16, zw, 0)

            pltpu.sync_copy(sbuf, out_hbm.at[pl.ds(cb, _CH), pl.ds(0, H)])
            pltpu.sync_copy(ebuf, out_hbm.at[pl.ds(cb, _CH), pl.ds(H, H)])
            pltpu.sync_copy(wbuf, out_hbm.at[pl.ds(cb, _CH), pl.ds(2 * H, WD)])
            return 0

        lax.fori_loop(0, nchunk, chunk_body, 0)

    return k(feat, wemb, gs, ge, wd, ends)


def kernel(features, clause_candidates, width_embedding):
    B, S, H = features.shape
    N = clause_candidates.shape[1]
    WD = width_embedding.shape[1]

    cc = clause_candidates.astype(jnp.int32)
    starts = cc[:, :, 0]
    ends = cc[:, :, 1]
    boff = (jnp.arange(B, dtype=jnp.int32) * S)[:, None]
    gs = (starts + boff).reshape(B * N)
    ge = (ends + boff).reshape(B * N)
    wd = (ends - starts).reshape(B * N)
    ends_flat = ends.reshape(B * N)

    out = _sc_span_gather(
        features.reshape(B * S, H), width_embedding,
        gs, ge, wd, ends_flat, H=H, WD=WD, n_total=B * N,
    )
    return out.reshape(B, N, 2 * H + WD)


# speedup vs baseline: 1.1041x; 1.1041x over previous
import functools

import jax
import jax.numpy as jnp
from jax import lax
from jax.experimental import pallas as pl
from jax.experimental.pallas import tpu as pltpu
from jax.experimental.pallas import tpu_sc as plsc

_NC = 2
_NS = 16
_NW = _NC * _NS

_CH = 16
_NBUF = 2


def _sc_span_gather(feat, wemb, idxp, *, S, H, WD, B, n_total):
    OUT_D = 2 * H + WD
    spw = n_total // _NW
    nchunk = spw // _CH
    wpb = (n_total // B) // spw
    mesh = plsc.VectorSubcoreMesh(core_axis_name="c", subcore_axis_name="s")

    @functools.partial(
        pl.kernel,
        out_type=jax.ShapeDtypeStruct((n_total, OUT_D), jnp.float32),
        mesh=mesh,
        compiler_params=pltpu.CompilerParams(use_tc_tiling_on_sc=False),
        scratch_types=[
            [pltpu.VMEM((3, _CH), jnp.int32) for _ in range(_NBUF)],
            [pltpu.VMEM((_CH, H), jnp.float32) for _ in range(_NBUF)],
            [pltpu.VMEM((_CH, H), jnp.float32) for _ in range(_NBUF)],
            [pltpu.VMEM((_CH, WD), jnp.float32) for _ in range(_NBUF)],
            [pltpu.SemaphoreType.DMA for _ in range(_NBUF)],
            [pltpu.SemaphoreType.DMA for _ in range(_NBUF)],
            [pltpu.SemaphoreType.DMA for _ in range(_NBUF)],
        ],
    )
    def k(feat_hbm, wemb_hbm, idx_hbm, out_hbm, idxb, sb, eb, wb,
          semI, semG, semS):
        wid = lax.axis_index("s") * _NC + lax.axis_index("c")
        base = wid * spw
        b_s = (wid // wpb) * S

        def idx_desc(c, b):
            return pltpu.make_async_copy(
                idx_hbm.at[:, pl.ds(base + c * _CH, _CH)], idxb[b], semI[b])

        def scat_descs(c, b):
            cb = base + c * _CH
            return (
                pltpu.make_async_copy(
                    sb[b], out_hbm.at[pl.ds(cb, _CH), pl.ds(0, H)], semS[b]),
                pltpu.make_async_copy(
                    eb[b], out_hbm.at[pl.ds(cb, _CH), pl.ds(H, H)], semS[b]),
                pltpu.make_async_copy(
                    wb[b], out_hbm.at[pl.ds(cb, _CH), pl.ds(2 * H, WD)],
                    semS[b]),
            )

        for b in range(_NBUF):
            idx_desc(b, b).start()

        def sub(c, b):
            @pl.when(c >= _NBUF)
            def _():
                for d in scat_descs(c, b):
                    d.wait()

            idx_desc(c, b).wait()
            g0 = pltpu.async_copy(feat_hbm.at[idxb[b].at[0]], sb[b], semG[b])
            g1 = pltpu.async_copy(feat_hbm.at[idxb[b].at[1]], eb[b], semG[b])
            g2 = pltpu.async_copy(wemb_hbm.at[idxb[b].at[2]], wb[b], semG[b])
            g0.wait()
            g1.wait()
            g2.wait()

            @pl.when(c + _NBUF < nchunk)
            def _():
                idx_desc(c + _NBUF, b).start()

            evec = idxb[b][1, :] - b_s
            z = jnp.zeros((16,), jnp.float32)
            for j in range(_CH):
                @pl.when(evec[j] <= 0)
                def _zero(j=j, b=b):
                    def zrow(kk, _):
                        sb[b][j, pl.ds(kk * 16, 16)] = z
                        eb[b][j, pl.ds(kk * 16, 16)] = z
                        return 0
                    lax.fori_loop(0, H // 16, zrow, 0)

                    def zw(kk, _):
                        wb[b][j, pl.ds(kk * 16, 16)] = z
                        return 0
                    lax.fori_loop(0, WD // 16, zw, 0)

            for d in scat_descs(c, b):
                d.start()

        def body(t, _):
            for b in range(_NBUF):
                sub(t * _NBUF + b, b)
            return 0

        lax.fori_loop(0, nchunk // _NBUF, body, 0)
        for b in range(_NBUF):
            for d in scat_descs(0, b):
                d.wait()

    return k(feat, wemb, idxp)


def kernel(features, clause_candidates, width_embedding):
    B, S, H = features.shape
    N = clause_candidates.shape[1]
    WD = width_embedding.shape[1]

    cc = clause_candidates.astype(jnp.int32)
    starts = cc[:, :, 0]
    ends = cc[:, :, 1]
    boff = (jnp.arange(B, dtype=jnp.int32) * S)[:, None]
    idxp = jnp.stack([
        (starts + boff).reshape(B * N),
        (ends + boff).reshape(B * N),
        (ends - starts).reshape(B * N),
    ])

    out = _sc_span_gather(
        features.reshape(B * S, H), width_embedding, idxp,
        S=S, H=H, WD=WD, B=B, n_total=B * N,
    )
    return out.reshape(B, N, 2 * H + WD)


# speedup vs baseline: 2.1451x; 1.9428x over previous
import functools

import jax
import jax.numpy as jnp
from jax import lax
from jax.experimental import pallas as pl
from jax.experimental.pallas import tpu as pltpu
from jax.experimental.pallas import tpu_sc as plsc

_NC = 2
_NS = 16
_NW = _NC * _NS

_CH = 16
_NBUF = 2


def _sc_span_gather(feat, wemb2, idxp, *, S, H, WD, B, n_total):
    OUT_D = 2 * H + WD
    spw = n_total // _NW
    nchunk = spw // _CH
    wpb = (n_total // B) // spw
    mesh = plsc.VectorSubcoreMesh(core_axis_name="c", subcore_axis_name="s")

    @functools.partial(
        pl.kernel,
        out_type=jax.ShapeDtypeStruct((n_total, OUT_D), jnp.float32),
        mesh=mesh,
        compiler_params=pltpu.CompilerParams(use_tc_tiling_on_sc=True),
        scratch_types=[
            pltpu.VMEM((3, spw), jnp.int32),
            [pltpu.VMEM((_CH,), jnp.int32) for _ in range(_NBUF)],
            [pltpu.VMEM((_CH, OUT_D), jnp.float32) for _ in range(_NBUF)],
            [pltpu.VMEM((_CH, 2 * WD), jnp.float32) for _ in range(_NBUF)],
            [pltpu.SemaphoreType.DMA for _ in range(_NBUF)],
            [pltpu.SemaphoreType.DMA for _ in range(_NBUF)],
        ],
    )
    def k(feat_hbm, wemb_hbm, idx_hbm, out_hbm, idxw, widx, obuf, wbuf,
          semG, semS):
        wid = lax.axis_index("s") * _NC + lax.axis_index("c")
        base = wid * spw
        b_s = (wid // wpb) * S

        pltpu.sync_copy(idx_hbm.at[:, pl.ds(base, spw)], idxw)

        def scat_desc(c, b):
            return pltpu.make_async_copy(
                obuf[b], out_hbm.at[pl.ds(base + c * _CH, _CH)], semS[b])

        def sub(c, b):
            @pl.when(c >= _NBUF)
            def _():
                scat_desc(c, b).wait()

            co = c * _CH
            wvec = idxw[2, pl.ds(co, _CH)]
            widx[b][...] = lax.shift_right_logical(wvec, 1)
            g0 = pltpu.async_copy(
                feat_hbm.at[idxw.at[0, pl.ds(co, _CH)]],
                obuf[b].at[:, pl.ds(0, H)], semG[b])
            g1 = pltpu.async_copy(
                feat_hbm.at[idxw.at[1, pl.ds(co, _CH)]],
                obuf[b].at[:, pl.ds(H, H)], semG[b])
            g2 = pltpu.async_copy(wemb_hbm.at[widx[b]], wbuf[b], semG[b])
            g0.wait()
            g1.wait()
            g2.wait()

            evec = idxw[1, pl.ds(co, _CH)] - b_s
            z = jnp.zeros((16,), jnp.float32)
            for j in range(_CH):
                off = (wvec[j] & 1) * WD
                for kk in range(WD // 16):
                    obuf[b][j, pl.ds(2 * H + kk * 16, 16)] = (
                        wbuf[b][j, pl.ds(off + kk * 16, 16)])

                @pl.when(evec[j] <= 0)
                def _zero(j=j, b=b):
                    def zrow(kk, _):
                        obuf[b][j, pl.ds(kk * 16, 16)] = z
                        return 0
                    lax.fori_loop(0, OUT_D // 16, zrow, 0)

            scat_desc(c, b).start()

        def body(t, _):
            for b in range(_NBUF):
                sub(t * _NBUF + b, b)
            return 0

        lax.fori_loop(0, nchunk // _NBUF, body, 0)
        for b in range(_NBUF):
            scat_desc(0, b).wait()

    return k(feat, wemb2, idxp)


def kernel(features, clause_candidates, width_embedding):
    B, S, H = features.shape
    N = clause_candidates.shape[1]
    WD = width_embedding.shape[1]

    cc = clause_candidates.astype(jnp.int32)
    starts = cc[:, :, 0]
    ends = cc[:, :, 1]
    boff = (jnp.arange(B, dtype=jnp.int32) * S)[:, None]
    idxp = jnp.stack([
        (starts + boff).reshape(B * N),
        (ends + boff).reshape(B * N),
        (ends - starts).reshape(B * N),
    ])

    out = _sc_span_gather(
        features.reshape(B * S, H),
        width_embedding.reshape(-1, 2 * WD),
        idxp, S=S, H=H, WD=WD, B=B, n_total=B * N,
    )
    return out.reshape(B, N, 2 * H + WD)
